# baseline (speedup 1.0000x reference)
import functools

import jax
import jax.numpy as jnp
from jax import lax
from jax.experimental import pallas as pl
from jax.experimental.pallas import tpu as pltpu
from jax.experimental.pallas import tpu_sc as plsc

_N = 10000
_E = 320000
_D = 128
_NC = 2
_NS = 16
_NW = _NC * _NS
_CHUNK = 125
_NCHUNK = _E // _NW // _CHUNK
_NPAD = 10240
_RPT = _NPAD // _NS
_ROWBLK = 1000
_GRID = _N // _ROWBLK


def _sc_segsum(u, e3, zrows):
    mesh = plsc.VectorSubcoreMesh(core_axis_name="c", subcore_axis_name="s")

    @functools.partial(
        pl.kernel,
        mesh=mesh,
        out_type=jax.ShapeDtypeStruct((_NC, _NPAD, _D), jnp.float32),
        scratch_types=[
            pltpu.VMEM((2, _CHUNK), jnp.int32),
            pltpu.VMEM((2, _CHUNK), jnp.int32),
            pltpu.VMEM((_CHUNK, _D), jnp.float32),
            pltpu.VMEM((_CHUNK, _D), jnp.float32),
            pltpu.VMEM_SHARED((_NPAD, _D), jnp.float32),
            pltpu.SemaphoreType.DMA,
            pltpu.SemaphoreType.DMA,
            pltpu.SemaphoreType.DMA,
            pltpu.SemaphoreType.DMA,
        ],
    )
    def k(u_hbm, e_hbm, z_hbm, out_hbm,
          idx_a, idx_b, rows_a, rows_b, acc_sh, sem_ia, sem_ib, sem_a, sem_b):
        c = lax.axis_index("c")
        s = lax.axis_index("s")
        wid = s * _NC + c
        pltpu.sync_copy(z_hbm, acc_sh.at[pl.ds(s * _RPT, _RPT)])
        plsc.subcore_barrier()

        pltpu.async_copy(e_hbm.at[wid, 0], idx_a, sem_ia)
        pltpu.async_copy(e_hbm.at[wid, 1], idx_b, sem_ib)
        pltpu.make_async_copy(e_hbm.at[wid, 0], idx_a, sem_ia).wait()
        pltpu.async_copy(u_hbm.at[idx_a.at[0]], rows_a, sem_a)

        def body(i, carry):
            g = 2 * i
            pltpu.make_async_copy(e_hbm.at[wid, g + 1], idx_b, sem_ib).wait()
            pltpu.async_copy(u_hbm.at[idx_b.at[0]], rows_b, sem_b)
            pltpu.make_async_copy(u_hbm.at[idx_a.at[0]], rows_a, sem_a).wait()
            pltpu.sync_copy(rows_a, acc_sh.at[idx_a.at[1]], add=True)

            @pl.when(i + 1 < _NCHUNK // 2)
            def _():
                pltpu.async_copy(e_hbm.at[wid, g + 2], idx_a, sem_ia)

            pltpu.make_async_copy(u_hbm.at[idx_b.at[0]], rows_b, sem_b).wait()
            pltpu.sync_copy(rows_b, acc_sh.at[idx_b.at[1]], add=True)

            @pl.when(i + 1 < _NCHUNK // 2)
            def _():
                pltpu.make_async_copy(e_hbm.at[wid, g + 2], idx_a, sem_ia).wait()
                pltpu.async_copy(u_hbm.at[idx_a.at[0]], rows_a, sem_a)
                pltpu.async_copy(e_hbm.at[wid, g + 3], idx_b, sem_ib)

            return carry

        lax.fori_loop(0, _NCHUNK // 2, body, 0)
        plsc.subcore_barrier()
        pltpu.sync_copy(
            acc_sh.at[pl.ds(s * _RPT, _RPT)],
            out_hbm.at[c, pl.ds(s * _RPT, _RPT)],
        )

    return k(u, e3, zrows)


def _sc_deg(ones, dst3, zrows):
    mesh = plsc.VectorSubcoreMesh(core_axis_name="c", subcore_axis_name="s")

    @functools.partial(
        pl.kernel,
        mesh=mesh,
        out_type=jax.ShapeDtypeStruct((_NC, _NPAD, _D), jnp.float32),
        scratch_types=[
            pltpu.VMEM((_NCHUNK, _CHUNK), jnp.int32),
            pltpu.VMEM((_CHUNK, _D), jnp.float32),
            pltpu.VMEM_SHARED((_NPAD, _D), jnp.float32),
        ],
    )
    def k(u_hbm, dst_hbm, z_hbm, out_hbm, idx_d, rows_a, acc_sh):
        c = lax.axis_index("c")
        s = lax.axis_index("s")
        wid = s * _NC + c
        pltpu.sync_copy(z_hbm, acc_sh.at[pl.ds(s * _RPT, _RPT)])
        pltpu.sync_copy(dst_hbm.at[wid], idx_d)
        pltpu.sync_copy(u_hbm, rows_a)
        plsc.subcore_barrier()

        def body(j, carry):
            pltpu.sync_copy(rows_a, acc_sh.at[idx_d.at[j]], add=True)
            return carry

        lax.fori_loop(0, _NCHUNK, body, 0)
        plsc.subcore_barrier()
        pltpu.sync_copy(
            acc_sh.at[pl.ds(s * _RPT, _RPT)],
            out_hbm.at[c, pl.ds(s * _RPT, _RPT)],
        )

    return k(ones, dst3, zrows)


def _dot(a, b):
    return jnp.dot(a, b, preferred_element_type=jnp.float32,
                   precision=lax.Precision.HIGHEST)


def _tc_mm(x, w):
    def body(x_ref, w_ref, o_ref):
        o_ref[...] = _dot(x_ref[...], w_ref[...])

    return pl.pallas_call(
        body,
        grid=(_GRID,),
        in_specs=[
            pl.BlockSpec((_ROWBLK, _D), lambda i: (i, 0)),
            pl.BlockSpec((_D, 2 * _D), lambda i: (0, 0)),
        ],
        out_specs=pl.BlockSpec((_ROWBLK, 2 * _D), lambda i: (i, 0)),
        out_shape=jax.ShapeDtypeStruct((_N, 2 * _D), jnp.float32),
    )(x, w)


def _tc_fuse(a0, a1, r, v, b, w):
    def body(a0_ref, a1_ref, r_ref, v_ref, b_ref, w_ref, o_ref):
        h = jnp.maximum(
            (a0_ref[...] + a1_ref[...]) * r_ref[...] + b_ref[...] + v_ref[...],
            0.0,
        )
        o_ref[...] = _dot(h, w_ref[...])

    blk = lambda: pl.BlockSpec((_ROWBLK, _D), lambda i: (i, 0))
    return pl.pallas_call(
        body,
        grid=(_GRID,),
        in_specs=[
            blk(), blk(), blk(), blk(),
            pl.BlockSpec((1, _D), lambda i: (0, 0)),
            pl.BlockSpec((_D, 2 * _D), lambda i: (0, 0)),
        ],
        out_specs=pl.BlockSpec((_ROWBLK, 2 * _D), lambda i: (i, 0)),
        out_shape=jax.ShapeDtypeStruct((_N, 2 * _D), jnp.float32),
    )(a0, a1, r, v, b, w)


def _tc_final(a0, a1, r, v, b):
    def body(a0_ref, a1_ref, r_ref, v_ref, b_ref, o_ref):
        o_ref[...] = (
            (a0_ref[...] + a1_ref[...]) * r_ref[...] + b_ref[...] + v_ref[...]
        )

    blk = lambda: pl.BlockSpec((_ROWBLK, _D), lambda i: (i, 0))
    return pl.pallas_call(
        body,
        grid=(_GRID,),
        in_specs=[blk(), blk(), blk(), blk(),
                  pl.BlockSpec((1, _D), lambda i: (0, 0))],
        out_specs=blk(),
        out_shape=jax.ShapeDtypeStruct((_N, _D), jnp.float32),
    )(a0, a1, r, v, b)


def _tc_recip(d0, d1):
    def body(d0_ref, d1_ref, o_ref):
        o_ref[...] = 1.0 / jnp.maximum(d0_ref[...] + d1_ref[...], 1.0)

    blk = lambda: pl.BlockSpec((_ROWBLK, _D), lambda i: (i, 0))
    return pl.pallas_call(
        body,
        grid=(_GRID,),
        in_specs=[blk(), blk()],
        out_specs=blk(),
        out_shape=jax.ShapeDtypeStruct((_N, _D), jnp.float32),
    )(d0, d1)


def kernel(x, edge_index, Wl1, bl1, Wr1, Wl2, bl2, Wr2, Wl3, bl3, Wr3,
           Wl4, bl4, Wr4):
    src3 = edge_index[0].astype(jnp.int32).reshape(_NW, _NCHUNK, _CHUNK)
    dst3 = edge_index[1].astype(jnp.int32).reshape(_NW, _NCHUNK, _CHUNK)
    e3 = jnp.stack([src3, dst3], axis=2)
    zrows = jnp.zeros((_RPT, _D), jnp.float32)
    ones = jnp.ones((_CHUNK, _D), jnp.float32)

    degp = _sc_deg(ones, dst3, zrows)
    r = _tc_recip(degp[0, :_N], degp[1, :_N])

    w1 = jnp.concatenate([Wl1.T, Wr1.T], axis=1)
    w2 = jnp.concatenate([Wl2.T, Wr2.T], axis=1)
    w3 = jnp.concatenate([Wl3.T, Wr3.T], axis=1)
    w4 = jnp.concatenate([Wl4.T, Wr4.T], axis=1)
    b1 = bl1.reshape(1, _D)
    b2 = bl2.reshape(1, _D)
    b3 = bl3.reshape(1, _D)
    b4 = bl4.reshape(1, _D)

    t = _tc_mm(x, w1)
    p = _sc_segsum(t[:, :_D], e3, zrows)
    t = _tc_fuse(p[0, :_N], p[1, :_N], r, t[:, _D:], b1, w2)
    p = _sc_segsum(t[:, :_D], e3, zrows)
    t = _tc_fuse(p[0, :_N], p[1, :_N], r, t[:, _D:], b2, w3)
    p = _sc_segsum(t[:, :_D], e3, zrows)
    t = _tc_fuse(p[0, :_N], p[1, :_N], r, t[:, _D:], b3, w4)
    p = _sc_segsum(t[:, :_D], e3, zrows)
    return _tc_final(p[0, :_N], p[1, :_N], r, t[:, _D:], b4)


# speedup vs baseline: 1.0241x; 1.0241x over previous
import functools

import jax
import jax.numpy as jnp
from jax import lax
from jax.experimental import pallas as pl
from jax.experimental.pallas import tpu as pltpu
from jax.experimental.pallas import tpu_sc as plsc

_N = 10000
_E = 320000
_D = 128
_NC = 2
_NS = 16
_NW = _NC * _NS
_CHUNK = 50
_NCHUNK = _E // _NW // _CHUNK
_NSLOT = 4
_CPAD = 56
_NPAD = 10240
_RPT = _NPAD // _NS
_ROWBLK = 1000
_GRID = _N // _ROWBLK


def _sc_segsum(u, src3, dst3, zrows):
    mesh = plsc.VectorSubcoreMesh(core_axis_name="c", subcore_axis_name="s")

    @functools.partial(
        pl.kernel,
        mesh=mesh,
        out_type=jax.ShapeDtypeStruct((_NC, _NPAD, _D), jnp.float32),
        scratch_types=(
            [pltpu.VMEM((_NCHUNK * _CPAD,), jnp.int32),
             pltpu.VMEM((_NSLOT, _CHUNK), jnp.int32),
             pltpu.VMEM((_NSLOT, _CHUNK, _D), jnp.float32),
             pltpu.VMEM_SHARED((_NPAD, _D), jnp.float32)]
            + [pltpu.SemaphoreType.DMA for _ in range(3 * _NSLOT)]
        ),
    )
    def k(u_hbm, src_hbm, dst_hbm, z_hbm, out_hbm, idx_s, dst2, rows3,
          acc_sh, *sems):
        dsts = [dst2.at[k_] for k_ in range(_NSLOT)]
        rows = [rows3.at[k_] for k_ in range(_NSLOT)]
        sem_d = sems[:_NSLOT]
        sem_g = sems[_NSLOT:2 * _NSLOT]
        sem_s = sems[2 * _NSLOT:]
        c = lax.axis_index("c")
        s = lax.axis_index("s")
        wid = s * _NC + c
        pltpu.sync_copy(z_hbm, acc_sh.at[pl.ds(s * _RPT, _RPT)])
        pltpu.sync_copy(src_hbm.at[wid], idx_s)
        plsc.subcore_barrier()

        def srcidx(g):
            return idx_s.at[pl.ds(pl.multiple_of(g * _CPAD, 8), _CHUNK)]

        for k_ in range(_NSLOT):
            pltpu.async_copy(dst_hbm.at[wid, k_], dsts[k_], sem_d[k_])
            pltpu.async_copy(u_hbm.at[srcidx(k_)], rows[k_], sem_g[k_])

        def body(i, carry):
            for k_ in range(_NSLOT):
                g = _NSLOT * i + k_
                pltpu.make_async_copy(
                    u_hbm.at[srcidx(g)], rows[k_], sem_g[k_]).wait()
                pltpu.make_async_copy(
                    dst_hbm.at[wid, g], dsts[k_], sem_d[k_]).wait()
                pltpu.async_copy(
                    rows[k_], acc_sh.at[dsts[k_]], sem_s[k_], add=True)

                j = (k_ + 2) % _NSLOT
                gl = _NSLOT * i + k_ + 2

                @pl.when(jnp.logical_and(gl >= _NSLOT, gl < _NCHUNK))
                def _(j=j, gl=gl):
                    pltpu.make_async_copy(
                        rows[j], acc_sh.at[dsts[j]], sem_s[j]).wait()
                    pltpu.async_copy(dst_hbm.at[wid, gl], dsts[j], sem_d[j])
                    pltpu.async_copy(u_hbm.at[srcidx(gl)], rows[j], sem_g[j])

            return carry

        lax.fori_loop(0, _NCHUNK // _NSLOT, body, 0)
        for k_ in range(_NSLOT):
            pltpu.make_async_copy(
                rows[k_], acc_sh.at[dsts[k_]], sem_s[k_]).wait()
        plsc.subcore_barrier()
        pltpu.sync_copy(
            acc_sh.at[pl.ds(s * _RPT, _RPT)],
            out_hbm.at[c, pl.ds(s * _RPT, _RPT)],
        )

    return k(u, src3, dst3, zrows)


def _sc_deg(ones, dst3, zrows):
    mesh = plsc.VectorSubcoreMesh(core_axis_name="c", subcore_axis_name="s")

    @functools.partial(
        pl.kernel,
        mesh=mesh,
        out_type=jax.ShapeDtypeStruct((_NC, _NPAD, _D), jnp.float32),
        scratch_types=[
            pltpu.VMEM((_NCHUNK, _CHUNK), jnp.int32),
            pltpu.VMEM((_CHUNK, _D), jnp.float32),
            pltpu.VMEM_SHARED((_NPAD, _D), jnp.float32),
        ],
    )
    def k(u_hbm, dst_hbm, z_hbm, out_hbm, idx_d, rows_a, acc_sh):
        c = lax.axis_index("c")
        s = lax.axis_index("s")
        wid = s * _NC + c
        pltpu.sync_copy(z_hbm, acc_sh.at[pl.ds(s * _RPT, _RPT)])
        pltpu.sync_copy(dst_hbm.at[wid], idx_d)
        pltpu.sync_copy(u_hbm, rows_a)
        plsc.subcore_barrier()

        def body(j, carry):
            pltpu.sync_copy(rows_a, acc_sh.at[idx_d.at[j]], add=True)
            return carry

        lax.fori_loop(0, _NCHUNK, body, 0)
        plsc.subcore_barrier()
        pltpu.sync_copy(
            acc_sh.at[pl.ds(s * _RPT, _RPT)],
            out_hbm.at[c, pl.ds(s * _RPT, _RPT)],
        )

    return k(ones, dst3, zrows)


def _dot(a, b):
    return jnp.dot(a, b, preferred_element_type=jnp.float32,
                   precision=lax.Precision.HIGHEST)


def _tc_mm(x, w):
    def body(x_ref, w_ref, o_ref):
        o_ref[...] = _dot(x_ref[...], w_ref[...])

    return pl.pallas_call(
        body,
        grid=(_GRID,),
        in_specs=[
            pl.BlockSpec((_ROWBLK, _D), lambda i: (i, 0)),
            pl.BlockSpec((_D, 2 * _D), lambda i: (0, 0)),
        ],
        out_specs=pl.BlockSpec((_ROWBLK, 2 * _D), lambda i: (i, 0)),
        out_shape=jax.ShapeDtypeStruct((_N, 2 * _D), jnp.float32),
    )(x, w)


def _tc_fuse(a0, a1, r, v, b, w):
    def body(a0_ref, a1_ref, r_ref, v_ref, b_ref, w_ref, o_ref):
        h = jnp.maximum(
            (a0_ref[...] + a1_ref[...]) * r_ref[...] + b_ref[...] + v_ref[...],
            0.0,
        )
        o_ref[...] = _dot(h, w_ref[...])

    blk = lambda: pl.BlockSpec((_ROWBLK, _D), lambda i: (i, 0))
    return pl.pallas_call(
        body,
        grid=(_GRID,),
        in_specs=[
            blk(), blk(), blk(), blk(),
            pl.BlockSpec((1, _D), lambda i: (0, 0)),
            pl.BlockSpec((_D, 2 * _D), lambda i: (0, 0)),
        ],
        out_specs=pl.BlockSpec((_ROWBLK, 2 * _D), lambda i: (i, 0)),
        out_shape=jax.ShapeDtypeStruct((_N, 2 * _D), jnp.float32),
    )(a0, a1, r, v, b, w)


def _tc_final(a0, a1, r, v, b):
    def body(a0_ref, a1_ref, r_ref, v_ref, b_ref, o_ref):
        o_ref[...] = (
            (a0_ref[...] + a1_ref[...]) * r_ref[...] + b_ref[...] + v_ref[...]
        )

    blk = lambda: pl.BlockSpec((_ROWBLK, _D), lambda i: (i, 0))
    return pl.pallas_call(
        body,
        grid=(_GRID,),
        in_specs=[blk(), blk(), blk(), blk(),
                  pl.BlockSpec((1, _D), lambda i: (0, 0))],
        out_specs=blk(),
        out_shape=jax.ShapeDtypeStruct((_N, _D), jnp.float32),
    )(a0, a1, r, v, b)


def _tc_recip(d0, d1):
    def body(d0_ref, d1_ref, o_ref):
        o_ref[...] = 1.0 / jnp.maximum(d0_ref[...] + d1_ref[...], 1.0)

    blk = lambda: pl.BlockSpec((_ROWBLK, _D), lambda i: (i, 0))
    return pl.pallas_call(
        body,
        grid=(_GRID,),
        in_specs=[blk(), blk()],
        out_specs=blk(),
        out_shape=jax.ShapeDtypeStruct((_N, _D), jnp.float32),
    )(d0, d1)


def kernel(x, edge_index, Wl1, bl1, Wr1, Wl2, bl2, Wr2, Wl3, bl3, Wr3,
           Wl4, bl4, Wr4):
    src3 = edge_index[0].astype(jnp.int32).reshape(_NW, _NCHUNK, _CHUNK)
    dst3 = edge_index[1].astype(jnp.int32).reshape(_NW, _NCHUNK, _CHUNK)
    srcf = jnp.pad(src3, ((0, 0), (0, 0), (0, _CPAD - _CHUNK))
                   ).reshape(_NW, _NCHUNK * _CPAD)
    zrows = jnp.zeros((_RPT, _D), jnp.float32)
    ones = jnp.ones((_CHUNK, _D), jnp.float32)

    degp = _sc_deg(ones, dst3, zrows)
    r = _tc_recip(degp[0, :_N], degp[1, :_N])

    w1 = jnp.concatenate([Wl1.T, Wr1.T], axis=1)
    w2 = jnp.concatenate([Wl2.T, Wr2.T], axis=1)
    w3 = jnp.concatenate([Wl3.T, Wr3.T], axis=1)
    w4 = jnp.concatenate([Wl4.T, Wr4.T], axis=1)
    b1 = bl1.reshape(1, _D)
    b2 = bl2.reshape(1, _D)
    b3 = bl3.reshape(1, _D)
    b4 = bl4.reshape(1, _D)

    t = _tc_mm(x, w1)
    p = _sc_segsum(t[:, :_D], srcf, dst3, zrows)
    t = _tc_fuse(p[0, :_N], p[1, :_N], r, t[:, _D:], b1, w2)
    p = _sc_segsum(t[:, :_D], srcf, dst3, zrows)
    t = _tc_fuse(p[0, :_N], p[1, :_N], r, t[:, _D:], b2, w3)
    p = _sc_segsum(t[:, :_D], srcf, dst3, zrows)
    t = _tc_fuse(p[0, :_N], p[1, :_N], r, t[:, _D:], b3, w4)
    p = _sc_segsum(t[:, :_D], srcf, dst3, zrows)
    return _tc_final(p[0, :_N], p[1, :_N], r, t[:, _D:], b4)


# speedup vs baseline: 1.0969x; 1.0711x over previous
import functools

import jax
import jax.numpy as jnp
from jax import lax
from jax.experimental import pallas as pl
from jax.experimental.pallas import tpu as pltpu
from jax.experimental.pallas import tpu_sc as plsc

_N = 10000
_E = 320000
_D = 128
_NC = 2
_NS = 16
_NW = _NC * _NS
_CHUNK = 100
_NCHUNK = _E // _NW // _CHUNK
_NPAD = 10240
_RPT = _NPAD // _NS
_ROWBLK = 1000
_GRID = _N // _ROWBLK


def _sc_segsum(u, src3, dst3, zrows):
    mesh = plsc.VectorSubcoreMesh(core_axis_name="c", subcore_axis_name="s")

    @functools.partial(
        pl.kernel,
        mesh=mesh,
        out_type=jax.ShapeDtypeStruct((_NC, _NPAD, _D), jnp.float32),
        scratch_types=[
            pltpu.VMEM((_NCHUNK, _CHUNK), jnp.int32),
            pltpu.VMEM((_CHUNK,), jnp.int32),
            pltpu.VMEM((_CHUNK,), jnp.int32),
            pltpu.VMEM((_CHUNK, _D), jnp.float32),
            pltpu.VMEM((_CHUNK, _D), jnp.float32),
            pltpu.VMEM_SHARED((_NPAD, _D), jnp.float32),
            pltpu.SemaphoreType.DMA,
            pltpu.SemaphoreType.DMA,
            pltpu.SemaphoreType.DMA,
            pltpu.SemaphoreType.DMA,
        ],
    )
    def k(u_hbm, src_hbm, dst_hbm, z_hbm, out_hbm,
          idx_s, dst_a, dst_b, rows_a, rows_b, acc_sh,
          sem_a, sem_b, sem_da, sem_db):
        c = lax.axis_index("c")
        s = lax.axis_index("s")
        wid = s * _NC + c
        pltpu.sync_copy(z_hbm, acc_sh.at[pl.ds(s * _RPT, _RPT)])
        pltpu.sync_copy(src_hbm.at[wid], idx_s)
        plsc.subcore_barrier()

        pltpu.async_copy(dst_hbm.at[wid, 0], dst_a, sem_da)
        pltpu.async_copy(u_hbm.at[idx_s.at[0]], rows_a, sem_a)

        def body(i, carry):
            g = 2 * i
            pltpu.async_copy(dst_hbm.at[wid, g + 1], dst_b, sem_db)
            pltpu.async_copy(u_hbm.at[idx_s.at[g + 1]], rows_b, sem_b)
            pltpu.make_async_copy(u_hbm.at[idx_s.at[g]], rows_a, sem_a).wait()
            pltpu.make_async_copy(dst_hbm.at[wid, g], dst_a, sem_da).wait()
            pltpu.sync_copy(rows_a, acc_sh.at[dst_a], add=True)

            @pl.when(i + 1 < _NCHUNK // 2)
            def _():
                pltpu.async_copy(dst_hbm.at[wid, g + 2], dst_a, sem_da)
                pltpu.async_copy(u_hbm.at[idx_s.at[g + 2]], rows_a, sem_a)

            pltpu.make_async_copy(u_hbm.at[idx_s.at[g + 1]], rows_b, sem_b).wait()
            pltpu.make_async_copy(dst_hbm.at[wid, g + 1], dst_b, sem_db).wait()
            pltpu.sync_copy(rows_b, acc_sh.at[dst_b], add=True)
            return carry

        lax.fori_loop(0, _NCHUNK // 2, body, 0)
        plsc.subcore_barrier()
        pltpu.sync_copy(
            acc_sh.at[pl.ds(s * _RPT, _RPT)],
            out_hbm.at[c, pl.ds(s * _RPT, _RPT)],
        )

    return k(u, src3, dst3, zrows)


def _sc_deg(ones, dst3, zrows):
    mesh = plsc.VectorSubcoreMesh(core_axis_name="c", subcore_axis_name="s")

    @functools.partial(
        pl.kernel,
        mesh=mesh,
        out_type=jax.ShapeDtypeStruct((_NC, _NPAD, _D), jnp.float32),
        scratch_types=[
            pltpu.VMEM((_NCHUNK, _CHUNK), jnp.int32),
            pltpu.VMEM((_CHUNK, _D), jnp.float32),
            pltpu.VMEM_SHARED((_NPAD, _D), jnp.float32),
        ],
    )
    def k(u_hbm, dst_hbm, z_hbm, out_hbm, idx_d, rows_a, acc_sh):
        c = lax.axis_index("c")
        s = lax.axis_index("s")
        wid = s * _NC + c
        pltpu.sync_copy(z_hbm, acc_sh.at[pl.ds(s * _RPT, _RPT)])
        pltpu.sync_copy(dst_hbm.at[wid], idx_d)
        pltpu.sync_copy(u_hbm, rows_a)
        plsc.subcore_barrier()

        def body(j, carry):
            pltpu.sync_copy(rows_a, acc_sh.at[idx_d.at[j]], add=True)
            return carry

        lax.fori_loop(0, _NCHUNK, body, 0)
        plsc.subcore_barrier()
        pltpu.sync_copy(
            acc_sh.at[pl.ds(s * _RPT, _RPT)],
            out_hbm.at[c, pl.ds(s * _RPT, _RPT)],
        )

    return k(ones, dst3, zrows)


def _dot(a, b):
    return jnp.dot(a, b, preferred_element_type=jnp.float32,
                   precision=lax.Precision.HIGHEST)


def _tc_mm(x, w, d0, d1):
    def body(x_ref, w_ref, d0_ref, d1_ref, o_ref, r_ref):
        o_ref[...] = _dot(x_ref[...], w_ref[...])
        r_ref[...] = 1.0 / jnp.maximum(d0_ref[...] + d1_ref[...], 1.0)

    blk = lambda: pl.BlockSpec((_ROWBLK, _D), lambda i: (i, 0))
    return pl.pallas_call(
        body,
        grid=(_GRID,),
        in_specs=[
            blk(),
            pl.BlockSpec((_D, 2 * _D), lambda i: (0, 0)),
            blk(), blk(),
        ],
        out_specs=[pl.BlockSpec((_ROWBLK, 2 * _D), lambda i: (i, 0)), blk()],
        out_shape=[jax.ShapeDtypeStruct((_N, 2 * _D), jnp.float32),
                   jax.ShapeDtypeStruct((_N, _D), jnp.float32)],
    )(x, w, d0, d1)


def _tc_fuse(a0, a1, r, v, b, w):
    def body(a0_ref, a1_ref, r_ref, v_ref, b_ref, w_ref, o_ref):
        h = jnp.maximum(
            (a0_ref[...] + a1_ref[...]) * r_ref[...] + b_ref[...] + v_ref[...],
            0.0,
        )
        o_ref[...] = _dot(h, w_ref[...])

    blk = lambda: pl.BlockSpec((_ROWBLK, _D), lambda i: (i, 0))
    return pl.pallas_call(
        body,
        grid=(_GRID,),
        in_specs=[
            blk(), blk(), blk(), blk(),
            pl.BlockSpec((1, _D), lambda i: (0, 0)),
            pl.BlockSpec((_D, 2 * _D), lambda i: (0, 0)),
        ],
        out_specs=pl.BlockSpec((_ROWBLK, 2 * _D), lambda i: (i, 0)),
        out_shape=jax.ShapeDtypeStruct((_N, 2 * _D), jnp.float32),
    )(a0, a1, r, v, b, w)


def _tc_final(a0, a1, r, v, b):
    def body(a0_ref, a1_ref, r_ref, v_ref, b_ref, o_ref):
        o_ref[...] = (
            (a0_ref[...] + a1_ref[...]) * r_ref[...] + b_ref[...] + v_ref[...]
        )

    blk = lambda: pl.BlockSpec((_ROWBLK, _D), lambda i: (i, 0))
    return pl.pallas_call(
        body,
        grid=(_GRID,),
        in_specs=[blk(), blk(), blk(), blk(),
                  pl.BlockSpec((1, _D), lambda i: (0, 0))],
        out_specs=blk(),
        out_shape=jax.ShapeDtypeStruct((_N, _D), jnp.float32),
    )(a0, a1, r, v, b)


def kernel(x, edge_index, Wl1, bl1, Wr1, Wl2, bl2, Wr2, Wl3, bl3, Wr3,
           Wl4, bl4, Wr4):
    src3 = edge_index[0].astype(jnp.int32).reshape(_NW, _NCHUNK, _CHUNK)
    dst3 = edge_index[1].astype(jnp.int32).reshape(_NW, _NCHUNK, _CHUNK)
    zrows = jnp.zeros((_RPT, _D), jnp.float32)
    ones = jnp.ones((_CHUNK, _D), jnp.float32)

    degp = _sc_deg(ones, dst3, zrows)

    w1 = jnp.concatenate([Wl1.T, Wr1.T], axis=1)
    w2 = jnp.concatenate([Wl2.T, Wr2.T], axis=1)
    w3 = jnp.concatenate([Wl3.T, Wr3.T], axis=1)
    w4 = jnp.concatenate([Wl4.T, Wr4.T], axis=1)
    b1 = bl1.reshape(1, _D)
    b2 = bl2.reshape(1, _D)
    b3 = bl3.reshape(1, _D)
    b4 = bl4.reshape(1, _D)

    t, r = _tc_mm(x, w1, degp[0, :_N], degp[1, :_N])
    p = _sc_segsum(t[:, :_D], src3, dst3, zrows)
    t = _tc_fuse(p[0, :_N], p[1, :_N], r, t[:, _D:], b1, w2)
    p = _sc_segsum(t[:, :_D], src3, dst3, zrows)
    t = _tc_fuse(p[0, :_N], p[1, :_N], r, t[:, _D:], b2, w3)
    p = _sc_segsum(t[:, :_D], src3, dst3, zrows)
    t = _tc_fuse(p[0, :_N], p[1, :_N], r, t[:, _D:], b3, w4)
    p = _sc_segsum(t[:, :_D], src3, dst3, zrows)
    return _tc_final(p[0, :_N], p[1, :_N], r, t[:, _D:], b4)


# speedup vs baseline: 1.1382x; 1.0376x over previous
import functools

import jax
import jax.numpy as jnp
from jax import lax
from jax.experimental import pallas as pl
from jax.experimental.pallas import tpu as pltpu
from jax.experimental.pallas import tpu_sc as plsc

_N = 10000
_E = 320000
_D = 128
_NC = 2
_NS = 16
_NW = _NC * _NS
_CHUNK = 100
_NCHUNK = _E // _NW // _CHUNK
_NPAD = 10240
_RPT = _NPAD // _NS
_ROWBLK = 1000
_GRID = _N // _ROWBLK


def _sc_segsum(u, src3, dst3, zrows):
    mesh = plsc.VectorSubcoreMesh(core_axis_name="c", subcore_axis_name="s")

    @functools.partial(
        pl.kernel,
        mesh=mesh,
        out_type=jax.ShapeDtypeStruct((_NC, _NPAD, _D), jnp.float32),
        scratch_types=[
            pltpu.VMEM((_NCHUNK, _CHUNK), jnp.int32),
            pltpu.VMEM((_CHUNK,), jnp.int32),
            pltpu.VMEM((_CHUNK,), jnp.int32),
            pltpu.VMEM((_CHUNK, _D), jnp.float32),
            pltpu.VMEM((_CHUNK, _D), jnp.float32),
            pltpu.VMEM_SHARED((_NPAD, _D), jnp.float32),
            pltpu.SemaphoreType.DMA,
            pltpu.SemaphoreType.DMA,
            pltpu.SemaphoreType.DMA,
            pltpu.SemaphoreType.DMA,
        ],
    )
    def k(u_hbm, src_hbm, dst_hbm, z_hbm, out_hbm,
          idx_s, dst_a, dst_b, rows_a, rows_b, acc_sh,
          sem_a, sem_b, sem_da, sem_db):
        c = lax.axis_index("c")
        s = lax.axis_index("s")
        wid = s * _NC + c
        pltpu.sync_copy(z_hbm, acc_sh.at[pl.ds(s * _RPT, _RPT)])
        pltpu.sync_copy(src_hbm.at[wid], idx_s)
        plsc.subcore_barrier()

        pltpu.async_copy(dst_hbm.at[wid, 0], dst_a, sem_da)
        pltpu.async_copy(u_hbm.at[idx_s.at[0]], rows_a, sem_a)

        def body(i, carry):
            g = 2 * i
            pltpu.async_copy(dst_hbm.at[wid, g + 1], dst_b, sem_db)
            pltpu.async_copy(u_hbm.at[idx_s.at[g + 1]], rows_b, sem_b)
            pltpu.make_async_copy(u_hbm.at[idx_s.at[g]], rows_a, sem_a).wait()
            pltpu.make_async_copy(dst_hbm.at[wid, g], dst_a, sem_da).wait()
            pltpu.sync_copy(rows_a, acc_sh.at[dst_a], add=True)

            @pl.when(i + 1 < _NCHUNK // 2)
            def _():
                pltpu.async_copy(dst_hbm.at[wid, g + 2], dst_a, sem_da)
                pltpu.async_copy(u_hbm.at[idx_s.at[g + 2]], rows_a, sem_a)

            pltpu.make_async_copy(u_hbm.at[idx_s.at[g + 1]], rows_b, sem_b).wait()
            pltpu.make_async_copy(dst_hbm.at[wid, g + 1], dst_b, sem_db).wait()
            pltpu.sync_copy(rows_b, acc_sh.at[dst_b], add=True)
            return carry

        lax.fori_loop(0, _NCHUNK // 2, body, 0)
        plsc.subcore_barrier()
        pltpu.sync_copy(
            acc_sh.at[pl.ds(s * _RPT, _RPT)],
            out_hbm.at[c, pl.ds(s * _RPT, _RPT)],
        )

    return k(u, src3, dst3, zrows)


def _sc_deg(ones, dst3, zrows):
    mesh = plsc.VectorSubcoreMesh(core_axis_name="c", subcore_axis_name="s")

    @functools.partial(
        pl.kernel,
        mesh=mesh,
        out_type=jax.ShapeDtypeStruct((_NC, _NPAD, _D), jnp.float32),
        scratch_types=[
            pltpu.VMEM((_NCHUNK, _CHUNK), jnp.int32),
            pltpu.VMEM((_CHUNK, _D), jnp.float32),
            pltpu.VMEM_SHARED((_NPAD, _D), jnp.float32),
        ],
    )
    def k(u_hbm, dst_hbm, z_hbm, out_hbm, idx_d, rows_a, acc_sh):
        c = lax.axis_index("c")
        s = lax.axis_index("s")
        wid = s * _NC + c
        pltpu.sync_copy(z_hbm, acc_sh.at[pl.ds(s * _RPT, _RPT)])
        pltpu.sync_copy(dst_hbm.at[wid], idx_d)
        pltpu.sync_copy(u_hbm, rows_a)
        plsc.subcore_barrier()

        def body(j, carry):
            pltpu.sync_copy(rows_a, acc_sh.at[idx_d.at[j]], add=True)
            return carry

        lax.fori_loop(0, _NCHUNK, body, 0)
        plsc.subcore_barrier()
        pltpu.sync_copy(
            acc_sh.at[pl.ds(s * _RPT, _RPT)],
            out_hbm.at[c, pl.ds(s * _RPT, _RPT)],
        )

    return k(ones, dst3, zrows)


def _dot(a, b):
    return jnp.dot(a, b, preferred_element_type=jnp.float32,
                   precision=lax.Precision.HIGHEST)


def _tc_mm(x, w):
    def body(x_ref, w_ref, o_ref):
        o_ref[...] = _dot(x_ref[...], w_ref[...])

    return pl.pallas_call(
        body,
        grid=(_GRID,),
        in_specs=[
            pl.BlockSpec((_ROWBLK, _D), lambda i: (i, 0)),
            pl.BlockSpec((_D, 2 * _D), lambda i: (0, 0)),
        ],
        out_specs=pl.BlockSpec((_ROWBLK, 2 * _D), lambda i: (i, 0)),
        out_shape=jax.ShapeDtypeStruct((_N, 2 * _D), jnp.float32),
    )(x, w)


def _tc_recip(d0, d1):
    def body(d0_ref, d1_ref, o_ref):
        o_ref[...] = 1.0 / jnp.maximum(d0_ref[...] + d1_ref[...], 1.0)

    blk = lambda: pl.BlockSpec((_ROWBLK, _D), lambda i: (i, 0))
    return pl.pallas_call(
        body,
        grid=(_GRID,),
        in_specs=[blk(), blk()],
        out_specs=blk(),
        out_shape=jax.ShapeDtypeStruct((_N, _D), jnp.float32),
    )(d0, d1)


def _tc_fuse(a0, a1, r, v, b, w):
    def body(a0_ref, a1_ref, r_ref, v_ref, b_ref, w_ref, o_ref):
        h = jnp.maximum(
            (a0_ref[...] + a1_ref[...]) * r_ref[...] + b_ref[...] + v_ref[...],
            0.0,
        )
        o_ref[...] = _dot(h, w_ref[...])

    blk = lambda: pl.BlockSpec((_ROWBLK, _D), lambda i: (i, 0))
    return pl.pallas_call(
        body,
        grid=(_GRID,),
        in_specs=[
            blk(), blk(), blk(), blk(),
            pl.BlockSpec((1, _D), lambda i: (0, 0)),
            pl.BlockSpec((_D, 2 * _D), lambda i: (0, 0)),
        ],
        out_specs=pl.BlockSpec((_ROWBLK, 2 * _D), lambda i: (i, 0)),
        out_shape=jax.ShapeDtypeStruct((_N, 2 * _D), jnp.float32),
    )(a0, a1, r, v, b, w)


def _tc_final(a0, a1, r, v, b):
    def body(a0_ref, a1_ref, r_ref, v_ref, b_ref, o_ref):
        o_ref[...] = (
            (a0_ref[...] + a1_ref[...]) * r_ref[...] + b_ref[...] + v_ref[...]
        )

    blk = lambda: pl.BlockSpec((_ROWBLK, _D), lambda i: (i, 0))
    return pl.pallas_call(
        body,
        grid=(_GRID,),
        in_specs=[blk(), blk(), blk(), blk(),
                  pl.BlockSpec((1, _D), lambda i: (0, 0))],
        out_specs=blk(),
        out_shape=jax.ShapeDtypeStruct((_N, _D), jnp.float32),
    )(a0, a1, r, v, b)


def kernel(x, edge_index, Wl1, bl1, Wr1, Wl2, bl2, Wr2, Wl3, bl3, Wr3,
           Wl4, bl4, Wr4):
    src3 = edge_index[0].astype(jnp.int32).reshape(_NW, _NCHUNK, _CHUNK)
    dst3 = edge_index[1].astype(jnp.int32).reshape(_NW, _NCHUNK, _CHUNK)
    zrows = jnp.zeros((_RPT, _D), jnp.float32)
    ones = jnp.ones((_CHUNK, _D), jnp.float32)

    degp = _sc_deg(ones, dst3, zrows)
    r = _tc_recip(degp[0, :_N], degp[1, :_N])

    w1 = jnp.concatenate([Wl1.T, Wr1.T], axis=1)
    w2 = jnp.concatenate([Wl2.T, Wr2.T], axis=1)
    w3 = jnp.concatenate([Wl3.T, Wr3.T], axis=1)
    w4 = jnp.concatenate([Wl4.T, Wr4.T], axis=1)
    b1 = bl1.reshape(1, _D)
    b2 = bl2.reshape(1, _D)
    b3 = bl3.reshape(1, _D)
    b4 = bl4.reshape(1, _D)

    t = _tc_mm(x, w1)
    p = _sc_segsum(t[:, :_D], src3, dst3, zrows)
    t = _tc_fuse(p[0, :_N], p[1, :_N], r, t[:, _D:], b1, w2)
    p = _sc_segsum(t[:, :_D], src3, dst3, zrows)
    t = _tc_fuse(p[0, :_N], p[1, :_N], r, t[:, _D:], b2, w3)
    p = _sc_segsum(t[:, :_D], src3, dst3, zrows)
    t = _tc_fuse(p[0, :_N], p[1, :_N], r, t[:, _D:], b3, w4)
    p = _sc_segsum(t[:, :_D], src3, dst3, zrows)
    return _tc_final(p[0, :_N], p[1, :_N], r, t[:, _D:], b4)
